# baseline (speedup 1.0000x reference)
import jax
import jax.numpy as jnp
from jax.experimental import pallas as pl
from jax.experimental.pallas import tpu as pltpu

B, N, C_IN, C_OUT, EMB, RANK = 16, 512, 128, 128, 64, 16
_EPS = 1e-8


def _gcn_body(x_ref, mem_ref, phy_ref, wg_ref, bg_ref, zu_ref, zv_ref,
              wp_ref, bp_ref, wm_ref, bm_ref,
              out_ref, elbo_ref, adj_s, pn_s):
    i = pl.program_id(0)

    @pl.when(i == 0)
    def _init():
        phy = phy_ref[...]
        row = jax.lax.broadcasted_iota(jnp.int32, (N, N), 0)
        col = jax.lax.broadcasted_iota(jnp.int32, (N, N), 1)
        a_hat = phy + (row == col).astype(jnp.float32)
        deg_r = jnp.sum(a_hat, axis=1, keepdims=True)
        deg_c = jnp.sum(a_hat, axis=0, keepdims=True)
        sr = jax.lax.rsqrt(jnp.maximum(deg_r, 1.0))
        sc = jax.lax.rsqrt(jnp.maximum(deg_c, 1.0))
        adj_s[...] = sr * a_hat * sc

        logits = jax.lax.dot_general(
            zu_ref[...], zv_ref[...], (((1,), (1,)), ((), ())),
            preferred_element_type=jnp.float32)
        p = jax.nn.sigmoid(logits)
        pn_s[...] = p / (jnp.sum(p, axis=1, keepdims=True) + _EPS)

        recon = jnp.mean(phy * jnp.log(p + _EPS)
                         + (1.0 - phy) * jnp.log(1.0 - p + _EPS))
        kl = jnp.mean(p * jnp.log(p / 0.5 + _EPS)
                      + (1.0 - p) * jnp.log((1.0 - p) / 0.5 + _EPS))
        elbo_ref[...] = (recon - kl)[None, None]

    xb = x_ref[0]
    xg = jnp.dot(xb, wg_ref[...], preferred_element_type=jnp.float32)
    xp = jnp.dot(xb, wp_ref[...], preferred_element_type=jnp.float32)
    att = jnp.dot(adj_s[...], xg, preferred_element_type=jnp.float32) + bg_ref[...]
    agg = jnp.dot(pn_s[...], xp, preferred_element_type=jnp.float32) + bp_ref[...]
    gate = jax.nn.sigmoid(
        jnp.dot(mem_ref[0], wm_ref[...], preferred_element_type=jnp.float32)
        + bm_ref[...])
    out_ref[0] = att + gate * agg


def kernel(x, memory, phy_graph, W_gcn, b_gcn, Z_u, Z_v, W_pg, b_pg, W_mem, b_mem):
    bg = b_gcn.reshape(1, C_OUT)
    bp = b_pg.reshape(1, C_OUT)
    bm = b_mem.reshape(1, C_OUT)

    const = lambda shape: pl.BlockSpec(shape, lambda i: (0,) * len(shape))
    out, elbo = pl.pallas_call(
        _gcn_body,
        grid=(B,),
        in_specs=[
            pl.BlockSpec((1, N, C_IN), lambda i: (i, 0, 0)),
            pl.BlockSpec((1, N, EMB), lambda i: (i, 0, 0)),
            const((N, N)),
            const((C_IN, C_OUT)),
            const((1, C_OUT)),
            const((N, RANK)),
            const((N, RANK)),
            const((C_IN, C_OUT)),
            const((1, C_OUT)),
            const((EMB, C_OUT)),
            const((1, C_OUT)),
        ],
        out_specs=[
            pl.BlockSpec((1, N, C_OUT), lambda i: (i, 0, 0)),
            pl.BlockSpec((1, 1), lambda i: (0, 0)),
        ],
        out_shape=[
            jax.ShapeDtypeStruct((B, N, C_OUT), jnp.float32),
            jax.ShapeDtypeStruct((1, 1), jnp.float32),
        ],
        scratch_shapes=[
            pltpu.VMEM((N, N), jnp.float32),
            pltpu.VMEM((N, N), jnp.float32),
        ],
        compiler_params=pltpu.CompilerParams(
            dimension_semantics=("arbitrary",)),
    )(x, memory, phy_graph, W_gcn, bg, Z_u, Z_v, W_pg, bp, W_mem, bm)
    return out, elbo[0, 0]
